# baseline (speedup 1.0000x reference)
import functools

import jax
import jax.numpy as jnp
from jax import lax
from jax.experimental import pallas as pl
from jax.experimental.pallas import tpu as pltpu
from jax.experimental.pallas import tpu_sc as plsc

NC, NS = 2, 16
NW = NC * NS
CHUNK = 128
HID = 64
LANES = 16


def _silu(x):
    return x * jax.nn.sigmoid(x)


def _ln(h, g, b, eps=1e-5):
    mu = jnp.mean(h, axis=-1, keepdims=True)
    var = jnp.mean((h - mu) ** 2, axis=-1, keepdims=True)
    return (h - mu) / jnp.sqrt(var + eps) * g + b


KBUF = 12


def _make_gather(n_rows, e_half, d, ebase):
    per_w = e_half // NW
    n_chunks = per_w // CHUNK
    n_groups = n_chunks // KBUF
    assert n_chunks % KBUF == 0
    mesh = plsc.VectorSubcoreMesh(core_axis_name="c", subcore_axis_name="s")

    @functools.partial(
        pl.kernel,
        out_type=jax.ShapeDtypeStruct((e_half, d), jnp.bfloat16),
        mesh=mesh,
        compiler_params=pltpu.CompilerParams(use_tc_tiling_on_sc=False),
        scratch_types=[
            pltpu.VMEM((KBUF, CHUNK), jnp.int32),
            pltpu.VMEM((KBUF, CHUNK, d), jnp.bfloat16),
            pltpu.SemaphoreType.DMA,
            pltpu.SemaphoreType.DMA,
            pltpu.SemaphoreType.DMA,
        ],
    )
    def gather_k(x_hbm, idx_hbm, out_hbm, idx_v, rows_v, sem_i, sem_g, sem_o):
        wid = lax.axis_index("s") * NC + lax.axis_index("c")
        base = wid * per_w

        def outer(g, carry):
            off0 = base + g * (KBUF * CHUNK)
            hi = [pltpu.async_copy(idx_hbm.at[pl.ds(ebase + off0 + b * CHUNK, CHUNK)],
                                   idx_v.at[b], sem_i) for b in range(KBUF)]
            hg = []
            for b in range(KBUF):
                hi[b].wait()
                hg.append(pltpu.async_copy(x_hbm.at[idx_v.at[b]],
                                           rows_v.at[b], sem_g))
            ho = []
            for b in range(KBUF):
                hg[b].wait()
                ho.append(pltpu.async_copy(
                    rows_v.at[b], out_hbm.at[pl.ds(off0 + b * CHUNK, CHUNK)],
                    sem_o))
            for h in ho:
                h.wait()
            return carry

        lax.fori_loop(0, n_groups, outer, 0)

    return gather_k


CW = 16
ZROWS = 640


def _make_scatter(e_half, n_pad, d, ebase):
    per_tile = e_half // NS
    n_chunks = per_tile // CHUNK
    n_groups = n_chunks // KBUF
    assert per_tile % CHUNK == 0 and n_chunks % KBUF == 0
    assert (n_pad // NS) % 8 == 0
    acc_rows = n_pad + NS * CHUNK
    zcopies = acc_rows // NS // ZROWS
    assert acc_rows % (NS * ZROWS) == 0
    nrb = KBUF
    mesh = plsc.VectorSubcoreMesh(core_axis_name="c", subcore_axis_name="s")

    scratch = [
        pltpu.VMEM((KBUF, CHUNK), jnp.int32),
        pltpu.VMEM((nrb, CHUNK, CW), jnp.float32),
        pltpu.VMEM_SHARED((acc_rows, CW), jnp.float32),
        pltpu.SemaphoreType.DMA,
        pltpu.SemaphoreType.DMA,
        pltpu.SemaphoreType.DMA,
    ]

    @functools.partial(
        pl.kernel,
        out_type=jax.ShapeDtypeStruct((n_pad, d), jnp.float32),
        mesh=mesh,
        compiler_params=pltpu.CompilerParams(use_tc_tiling_on_sc=False),
        scratch_types=scratch,
    )
    def scatter_k(msg_hbm, dst_hbm, zeros_hbm, out_hbm,
                  idx_v, rows_v, acc, sem_d, sem_m, sem_s):
        cid = lax.axis_index("c")
        sid = lax.axis_index("s")

        for p in range(2):
            pidx = cid * 2 + p
            col0 = cid * (2 * CW) + p * CW

            def zero_body(i, carry):
                pltpu.sync_copy(zeros_hbm,
                                acc.at[pl.ds((sid * zcopies + i) * ZROWS, ZROWS)])
                return carry

            lax.fori_loop(0, zcopies, zero_body, 0)
            plsc.subcore_barrier()

            def body(g, carry):
                off0 = (sid * n_chunks + g * KBUF) * CHUNK
                hd = [pltpu.async_copy(
                    dst_hbm.at[pl.ds(ebase + off0 + b * CHUNK, CHUNK)],
                    idx_v.at[b], sem_d) for b in range(KBUF)]
                hm = [pltpu.async_copy(
                    msg_hbm.at[pidx, pl.ds(off0 + b * CHUNK, CHUNK)],
                    rows_v.at[b], sem_m) for b in range(KBUF)]
                pending = []
                for b in range(KBUF):
                    hd[b].wait()
                    hm[b].wait()
                    pending.append(pltpu.async_copy(
                        rows_v.at[b], acc.at[idx_v.at[b]], sem_s, add=True))
                for h in pending:
                    h.wait()
                return carry

            lax.fori_loop(0, n_groups, body, 0)
            plsc.subcore_barrier()

            stripe = n_pad // NS
            pltpu.sync_copy(
                acc.at[pl.ds(sid * stripe, stripe)],
                out_hbm.at[pl.ds(sid * stripe, stripe), pl.ds(col0, CW)])
            plsc.subcore_barrier()

    return scatter_k


def _make_count(e_pad, n_pad):
    per_tile = e_pad // NW
    n_chunks = per_tile // CHUNK
    n_groups = n_chunks // KBUF
    acc_rows = n_pad + NS * CHUNK
    zcopies = acc_rows // NS // ZROWS
    mesh = plsc.VectorSubcoreMesh(core_axis_name="c", subcore_axis_name="s")

    scratch = [
        pltpu.VMEM((KBUF, CHUNK), jnp.int32),
        pltpu.VMEM((CHUNK, CW), jnp.float32),
        pltpu.VMEM_SHARED((acc_rows, CW), jnp.float32),
        pltpu.SemaphoreType.DMA,
        pltpu.SemaphoreType.DMA,
    ]

    @functools.partial(
        pl.kernel,
        out_type=jax.ShapeDtypeStruct((2, n_pad, CW), jnp.float32),
        mesh=mesh,
        compiler_params=pltpu.CompilerParams(use_tc_tiling_on_sc=False),
        scratch_types=scratch,
    )
    def count_k(dst_hbm, zeros_hbm, ones_hbm, out_hbm,
                idx_v, ones_v, acc, sem_d, sem_s):
        cid = lax.axis_index("c")
        sid = lax.axis_index("s")
        pltpu.sync_copy(ones_hbm, ones_v)

        def zero_body(i, carry):
            pltpu.sync_copy(zeros_hbm,
                            acc.at[pl.ds((sid * zcopies + i) * ZROWS, ZROWS)])
            return carry

        lax.fori_loop(0, zcopies, zero_body, 0)
        plsc.subcore_barrier()

        def body(g, carry):
            off0 = ((cid * NS + sid) * n_chunks + g * KBUF) * CHUNK
            hd = [pltpu.async_copy(
                dst_hbm.at[pl.ds(off0 + b * CHUNK, CHUNK)],
                idx_v.at[b], sem_d) for b in range(KBUF)]
            pending = []
            for b in range(KBUF):
                hd[b].wait()
                pending.append(pltpu.async_copy(
                    ones_v, acc.at[idx_v.at[b]], sem_s, add=True))
            for h in pending:
                h.wait()
            return carry

        lax.fori_loop(0, n_groups, body, 0)
        plsc.subcore_barrier()

        stripe = n_pad // NS
        pltpu.sync_copy(acc.at[pl.ds(sid * stripe, stripe)],
                        out_hbm.at[cid, pl.ds(sid * stripe, stripe)])
        plsc.subcore_barrier()

    return count_k


def _embed_nodes(atom_fea, emb_w, emb_b, g, b):
    n, f = atom_fea.shape
    blk = 2000
    grid = n // blk

    def body(a_ref, w_ref, b_ref, g_ref, bb_ref, o_ref, oh_ref):
        h = jnp.dot(a_ref[...], w_ref[...],
                    preferred_element_type=jnp.float32) + b_ref[...]
        y = _silu(_ln(h, g_ref[...], bb_ref[...]))
        o_ref[...] = y
        oh_ref[...] = y.astype(jnp.bfloat16)

    return pl.pallas_call(
        body,
        grid=(grid,),
        in_specs=[
            pl.BlockSpec((blk, f), lambda i: (i, 0)),
            pl.BlockSpec((f, HID), lambda i: (0, 0)),
            pl.BlockSpec((1, HID), lambda i: (0, 0)),
            pl.BlockSpec((1, HID), lambda i: (0, 0)),
            pl.BlockSpec((1, HID), lambda i: (0, 0)),
        ],
        out_specs=[pl.BlockSpec((blk, HID), lambda i: (i, 0)),
                   pl.BlockSpec((blk, HID), lambda i: (i, 0))],
        out_shape=[jax.ShapeDtypeStruct((n, HID), jnp.float32),
                   jax.ShapeDtypeStruct((n, HID), jnp.bfloat16)],
    )(atom_fea, emb_w, emb_b.reshape(1, -1), g.reshape(1, -1), b.reshape(1, -1))


def _embed_edges(nbr_fea_p, edge_w, edge_b):
    e_pad, f = nbr_fea_p.shape
    blk = 4096
    grid = e_pad // blk

    def body(a_ref, w_ref, b_ref, o_ref):
        h = jnp.dot(a_ref[...], w_ref[...],
                    preferred_element_type=jnp.float32) + b_ref[...]
        o_ref[...] = _silu(h).astype(jnp.bfloat16)

    return pl.pallas_call(
        body,
        grid=(grid,),
        in_specs=[
            pl.BlockSpec((blk, f), lambda i: (i, 0)),
            pl.BlockSpec((f, HID), lambda i: (0, 0)),
            pl.BlockSpec((1, HID), lambda i: (0, 0)),
        ],
        out_specs=pl.BlockSpec((blk, HID), lambda i: (i, 0)),
        out_shape=jax.ShapeDtypeStruct((e_pad, HID), jnp.bfloat16),
    )(nbr_fea_p, edge_w, edge_b.reshape(1, -1))


def _conv_mlp(gx, ef, w1x, w1e, b1, w2, b2, hoff):
    e_half = gx.shape[0]
    blk = 4096
    grid = e_half // blk

    def body(gx_ref, ef_ref, w1x_ref, w1e_ref, b1_ref, w2_ref, b2_ref, o_ref):
        h = (jnp.dot(gx_ref[...], w1x_ref[...], preferred_element_type=jnp.float32)
             + jnp.dot(ef_ref[...], w1e_ref[...], preferred_element_type=jnp.float32)
             + b1_ref[...])
        h = _silu(h)
        m = jnp.dot(h, w2_ref[...], preferred_element_type=jnp.float32) + b2_ref[...]
        m = _silu(m)
        for p in range(4):
            o_ref[p] = m[:, p * CW:(p + 1) * CW]

    return pl.pallas_call(
        body,
        grid=(grid,),
        in_specs=[
            pl.BlockSpec((blk, HID), lambda i: (i, 0)),
            pl.BlockSpec((blk, HID), lambda i: (i + hoff, 0)),
            pl.BlockSpec((HID, 2 * HID), lambda i: (0, 0)),
            pl.BlockSpec((HID, 2 * HID), lambda i: (0, 0)),
            pl.BlockSpec((1, 2 * HID), lambda i: (0, 0)),
            pl.BlockSpec((2 * HID, HID), lambda i: (0, 0)),
            pl.BlockSpec((1, HID), lambda i: (0, 0)),
        ],
        out_specs=pl.BlockSpec((4, blk, CW), lambda i: (0, i, 0)),
        out_shape=jax.ShapeDtypeStruct((4, e_half, CW), jnp.float32),
    )(gx, ef, w1x, w1e, b1.reshape(1, -1), w2, b2.reshape(1, -1))


def _ln_residual(x, aggrs, cnt2, g, b):
    n = x.shape[0]
    blk = 2000
    grid = n // blk
    na = len(aggrs)

    def body(*refs):
        x_ref = refs[0]
        a_refs = refs[1:1 + na]
        ca_ref, cb_ref, g_ref, b_ref, o_ref, oh_ref = refs[1 + na:]
        cnt = ca_ref[0] + cb_ref[0]
        inv = 1.0 / jnp.maximum(cnt[:, 0:1], 1.0)
        agg = a_refs[0][...]
        for a in a_refs[1:]:
            agg = agg + a[...]
        y = x_ref[...] + agg * inv
        z = _ln(y, g_ref[...], b_ref[...])
        o_ref[...] = z
        oh_ref[...] = z.astype(jnp.bfloat16)

    return pl.pallas_call(
        body,
        grid=(grid,),
        in_specs=[
            pl.BlockSpec((blk, HID), lambda i: (i, 0)),
        ] + [
            pl.BlockSpec((blk, HID), lambda i: (i, 0)) for _ in range(na)
        ] + [
            pl.BlockSpec((1, blk, CW), lambda i: (0, i, 0)),
            pl.BlockSpec((1, blk, CW), lambda i: (1, i, 0)),
            pl.BlockSpec((1, HID), lambda i: (0, 0)),
            pl.BlockSpec((1, HID), lambda i: (0, 0)),
        ],
        out_specs=[pl.BlockSpec((blk, HID), lambda i: (i, 0)),
                   pl.BlockSpec((blk, HID), lambda i: (i, 0))],
        out_shape=[jax.ShapeDtypeStruct((n, HID), jnp.float32),
                   jax.ShapeDtypeStruct((n, HID), jnp.bfloat16)],
    )(x, *aggrs, cnt2, cnt2, g.reshape(1, -1), b.reshape(1, -1))


def _pool_head(x, bm3, w1, b1, w2, b2, w3p, b3p, n_graphs):
    n = x.shape[0]
    blk = 2000
    grid = n // blk

    def body(x_ref, bm_ref, w1_ref, b1_ref, w2_ref, b2_ref, w3_ref, b3_ref,
             o_ref, acc_ref):
        i = pl.program_id(0)

        @pl.when(i == 0)
        def _():
            acc_ref[...] = jnp.zeros_like(acc_ref)

        bm = bm_ref[...].reshape(1, blk)
        gid = lax.broadcasted_iota(jnp.int32, (n_graphs, blk), 0)
        oh = (gid == bm).astype(jnp.float32)
        xa = jnp.concatenate(
            [x_ref[...], jnp.ones((blk, HID), jnp.float32)], axis=1)
        acc_ref[...] += jnp.dot(oh, xa, preferred_element_type=jnp.float32)

        @pl.when(i == grid - 1)
        def _():
            acc = acc_ref[...]
            crystal = acc[:, :HID] / jnp.maximum(acc[:, HID:], 1.0)
            h = _silu(jnp.dot(crystal, w1_ref[...],
                              preferred_element_type=jnp.float32) + b1_ref[...])
            h = _silu(jnp.dot(h, w2_ref[...],
                              preferred_element_type=jnp.float32) + b2_ref[...])
            o_ref[...] = jnp.dot(h, w3_ref[...],
                                 preferred_element_type=jnp.float32) + b3_ref[...]

    return pl.pallas_call(
        body,
        grid=(grid,),
        in_specs=[
            pl.BlockSpec((blk, HID), lambda i: (i, 0)),
            pl.BlockSpec((1, 1, blk), lambda i: (i, 0, 0)),
            pl.BlockSpec((HID, HID), lambda i: (0, 0)),
            pl.BlockSpec((1, HID), lambda i: (0, 0)),
            pl.BlockSpec((HID, HID // 2), lambda i: (0, 0)),
            pl.BlockSpec((1, HID // 2), lambda i: (0, 0)),
            pl.BlockSpec((HID // 2, 128), lambda i: (0, 0)),
            pl.BlockSpec((1, 128), lambda i: (0, 0)),
        ],
        out_specs=pl.BlockSpec((n_graphs, 128), lambda i: (0, 0)),
        out_shape=jax.ShapeDtypeStruct((n_graphs, 128), jnp.float32),
        scratch_shapes=[pltpu.VMEM((n_graphs, 2 * HID), jnp.float32)],
    )(x, bm3, w1, b1.reshape(1, -1), w2, b2.reshape(1, -1), w3p, b3p)


def kernel(atom_fea, nbr_fea, nbr_idx, batch_mapping, params):
    n, _ = atom_fea.shape
    e = nbr_fea.shape[0]
    n_layers = 5
    n_graphs = 256

    nhalf = 1
    e_quant = nhalf * NW * CHUNK * KBUF
    e_pad = ((e + e_quant - 1) // e_quant) * e_quant
    e_half = e_pad // nhalf
    n_quant = NS * CHUNK
    n_pad = ((n + n_quant - 1) // n_quant) * n_quant

    pad = e_pad - e
    nbr_fea_p = jnp.pad(nbr_fea, ((0, pad), (0, 0)))
    idx32 = nbr_idx.astype(jnp.int32)
    src_p = jnp.pad(idx32[:, 0], (0, pad))
    dst_p = jnp.pad(idx32[:, 1], (0, pad), constant_values=n_pad)

    gathers = [_make_gather(n, e_half, HID, h * e_half) for h in range(nhalf)]
    scatters = [_make_scatter(e_half, n_pad, HID, h * e_half) for h in range(nhalf)]
    count_k = _make_count(e_pad, n_pad)
    mlp_hoff = e_half // 4096

    zeros_blk = jnp.zeros((ZROWS, CW), jnp.float32)
    ones_blk = jnp.ones((CHUNK, CW), jnp.float32)

    x, xh = _embed_nodes(atom_fea, params['emb_W'], params['emb_b'],
                         params['emb_ln_g'], params['emb_ln_b'])
    ef = _embed_edges(nbr_fea_p, params['edge_W'], params['edge_b'])
    cnt2 = count_k(dst_p, zeros_blk, ones_blk)

    for l in range(n_layers):
        w1 = params[f'conv{l}_W1']
        aggr = []
        for h in range(nhalf):
            gx = gathers[h](xh, src_p)
            msgs = _conv_mlp(gx, ef,
                             w1[:HID].astype(jnp.bfloat16),
                             w1[HID:].astype(jnp.bfloat16),
                             params[f'conv{l}_b1'],
                             params[f'conv{l}_W2'], params[f'conv{l}_b2'],
                             h * mlp_hoff)
            aggr.append(scatters[h](msgs, dst_p, zeros_blk))
        x, xh = _ln_residual(x, aggr, cnt2,
                             params[f'ln{l}_g'], params[f'ln{l}_b'])

    bm3 = batch_mapping.astype(jnp.int32).reshape(n // 2000, 1, 2000)
    w3p = jnp.pad(params['out_W3'], ((0, 0), (0, 128 - params['out_W3'].shape[1])))
    b3p = jnp.pad(params['out_b3'], (0, 128 - params['out_b3'].shape[0])).reshape(1, -1)
    out = _pool_head(x, bm3, params['out_W1'], params['out_b1'],
                     params['out_W2'], params['out_b2'], w3p, b3p, n_graphs)
    return out[:, :params['out_W3'].shape[1]]


# speedup vs baseline: 1.4741x; 1.4741x over previous
import functools

import jax
import jax.numpy as jnp
from jax import lax
from jax.experimental import pallas as pl
from jax.experimental.pallas import tpu as pltpu
from jax.experimental.pallas import tpu_sc as plsc

NC, NS = 2, 16
NW = NC * NS
CHUNK = 128
HID = 64
LANES = 16


def _silu(x):
    return x * jax.nn.sigmoid(x)


def _ln(h, g, b, eps=1e-5):
    mu = jnp.mean(h, axis=-1, keepdims=True)
    var = jnp.mean((h - mu) ** 2, axis=-1, keepdims=True)
    return (h - mu) / jnp.sqrt(var + eps) * g + b


KBUF = 12


def _make_gather(n_rows, e_half, d, ebase):
    per_w = e_half // NW
    n_chunks = per_w // CHUNK
    n_groups = n_chunks // KBUF
    assert n_chunks % KBUF == 0
    mesh = plsc.VectorSubcoreMesh(core_axis_name="c", subcore_axis_name="s")

    @functools.partial(
        pl.kernel,
        out_type=jax.ShapeDtypeStruct((e_half, d), jnp.bfloat16),
        mesh=mesh,
        compiler_params=pltpu.CompilerParams(use_tc_tiling_on_sc=False),
        scratch_types=[
            pltpu.VMEM((KBUF, CHUNK), jnp.int32),
            pltpu.VMEM((KBUF, CHUNK, d), jnp.bfloat16),
            pltpu.SemaphoreType.DMA,
            pltpu.SemaphoreType.DMA,
            pltpu.SemaphoreType.DMA,
        ],
    )
    def gather_k(x_hbm, idx_hbm, out_hbm, idx_v, rows_v, sem_i, sem_g, sem_o):
        wid = lax.axis_index("s") * NC + lax.axis_index("c")
        base = wid * per_w

        def outer(g, carry):
            off0 = base + g * (KBUF * CHUNK)
            hi = [pltpu.async_copy(idx_hbm.at[pl.ds(ebase + off0 + b * CHUNK, CHUNK)],
                                   idx_v.at[b], sem_i) for b in range(KBUF)]
            hg = []
            for b in range(KBUF):
                hi[b].wait()
                hg.append(pltpu.async_copy(x_hbm.at[idx_v.at[b]],
                                           rows_v.at[b], sem_g))
            ho = []
            for b in range(KBUF):
                hg[b].wait()
                ho.append(pltpu.async_copy(
                    rows_v.at[b], out_hbm.at[pl.ds(off0 + b * CHUNK, CHUNK)],
                    sem_o))
            for h in ho:
                h.wait()
            return carry

        lax.fori_loop(0, n_groups, outer, 0)

    return gather_k


CW = 16
ZROWS = 640


def _make_scatter(e_half, n_pad, d, ebase):
    per_tile = e_half // NS
    n_chunks = per_tile // CHUNK
    n_groups = n_chunks // KBUF
    assert per_tile % CHUNK == 0 and n_chunks % KBUF == 0
    assert (n_pad // NS) % 8 == 0
    acc_rows = n_pad + NS * CHUNK
    zcopies = acc_rows // NS // ZROWS
    assert acc_rows % (NS * ZROWS) == 0
    nrb = KBUF
    mesh = plsc.VectorSubcoreMesh(core_axis_name="c", subcore_axis_name="s")

    scratch = [
        pltpu.VMEM((KBUF, CHUNK), jnp.int32),
        pltpu.VMEM((nrb, CHUNK, CW), jnp.float32),
        pltpu.VMEM_SHARED((acc_rows, CW), jnp.float32),
        pltpu.SemaphoreType.DMA,
        pltpu.SemaphoreType.DMA,
        pltpu.SemaphoreType.DMA,
    ]

    @functools.partial(
        pl.kernel,
        out_type=jax.ShapeDtypeStruct((n_pad, d), jnp.float32),
        mesh=mesh,
        compiler_params=pltpu.CompilerParams(use_tc_tiling_on_sc=False),
        scratch_types=scratch,
    )
    def scatter_k(msg_hbm, dst_hbm, zeros_hbm, out_hbm,
                  idx_v, rows_v, acc, sem_d, sem_m, sem_s):
        cid = lax.axis_index("c")
        sid = lax.axis_index("s")

        for p in range(2):
            col0 = cid * (2 * CW) + p * CW

            def zero_body(i, carry):
                pltpu.sync_copy(zeros_hbm,
                                acc.at[pl.ds((sid * zcopies + i) * ZROWS, ZROWS)])
                return carry

            lax.fori_loop(0, zcopies, zero_body, 0)
            plsc.subcore_barrier()

            def body(g, carry):
                off0 = (sid * n_chunks + g * KBUF) * CHUNK
                hd = [pltpu.async_copy(
                    dst_hbm.at[pl.ds(ebase + off0 + b * CHUNK, CHUNK)],
                    idx_v.at[b], sem_d) for b in range(KBUF)]
                hm = [pltpu.async_copy(
                    msg_hbm.at[pl.ds(off0 + b * CHUNK, CHUNK), pl.ds(col0, CW)],
                    rows_v.at[b], sem_m) for b in range(KBUF)]
                pending = []
                for b in range(KBUF):
                    hd[b].wait()
                    hm[b].wait()
                    pending.append(pltpu.async_copy(
                        rows_v.at[b], acc.at[idx_v.at[b]], sem_s, add=True))
                for h in pending:
                    h.wait()
                return carry

            lax.fori_loop(0, n_groups, body, 0)
            plsc.subcore_barrier()

            stripe = n_pad // NS
            pltpu.sync_copy(
                acc.at[pl.ds(sid * stripe, stripe)],
                out_hbm.at[pl.ds(sid * stripe, stripe), pl.ds(col0, CW)])
            plsc.subcore_barrier()

    return scatter_k


def _make_count(e_pad, n_pad):
    per_tile = e_pad // NW
    n_chunks = per_tile // CHUNK
    n_groups = n_chunks // KBUF
    acc_rows = n_pad + NS * CHUNK
    zcopies = acc_rows // NS // ZROWS
    mesh = plsc.VectorSubcoreMesh(core_axis_name="c", subcore_axis_name="s")

    scratch = [
        pltpu.VMEM((KBUF, CHUNK), jnp.int32),
        pltpu.VMEM((CHUNK, CW), jnp.float32),
        pltpu.VMEM_SHARED((acc_rows, CW), jnp.float32),
        pltpu.SemaphoreType.DMA,
        pltpu.SemaphoreType.DMA,
    ]

    @functools.partial(
        pl.kernel,
        out_type=jax.ShapeDtypeStruct((2, n_pad, CW), jnp.float32),
        mesh=mesh,
        compiler_params=pltpu.CompilerParams(use_tc_tiling_on_sc=False),
        scratch_types=scratch,
    )
    def count_k(dst_hbm, zeros_hbm, ones_hbm, out_hbm,
                idx_v, ones_v, acc, sem_d, sem_s):
        cid = lax.axis_index("c")
        sid = lax.axis_index("s")
        pltpu.sync_copy(ones_hbm, ones_v)

        def zero_body(i, carry):
            pltpu.sync_copy(zeros_hbm,
                            acc.at[pl.ds((sid * zcopies + i) * ZROWS, ZROWS)])
            return carry

        lax.fori_loop(0, zcopies, zero_body, 0)
        plsc.subcore_barrier()

        def body(g, carry):
            off0 = ((cid * NS + sid) * n_chunks + g * KBUF) * CHUNK
            hd = [pltpu.async_copy(
                dst_hbm.at[pl.ds(off0 + b * CHUNK, CHUNK)],
                idx_v.at[b], sem_d) for b in range(KBUF)]
            pending = []
            for b in range(KBUF):
                hd[b].wait()
                pending.append(pltpu.async_copy(
                    ones_v, acc.at[idx_v.at[b]], sem_s, add=True))
            for h in pending:
                h.wait()
            return carry

        lax.fori_loop(0, n_groups, body, 0)
        plsc.subcore_barrier()

        stripe = n_pad // NS
        pltpu.sync_copy(acc.at[pl.ds(sid * stripe, stripe)],
                        out_hbm.at[cid, pl.ds(sid * stripe, stripe)])
        plsc.subcore_barrier()

    return count_k


def _embed_nodes(atom_fea, emb_w, emb_b, g, b):
    n, f = atom_fea.shape
    blk = 2000
    grid = n // blk

    def body(a_ref, w_ref, b_ref, g_ref, bb_ref, o_ref, oh_ref):
        h = jnp.dot(a_ref[...], w_ref[...],
                    preferred_element_type=jnp.float32) + b_ref[...]
        y = _silu(_ln(h, g_ref[...], bb_ref[...]))
        o_ref[...] = y
        oh_ref[...] = y.astype(jnp.bfloat16)

    return pl.pallas_call(
        body,
        grid=(grid,),
        in_specs=[
            pl.BlockSpec((blk, f), lambda i: (i, 0)),
            pl.BlockSpec((f, HID), lambda i: (0, 0)),
            pl.BlockSpec((1, HID), lambda i: (0, 0)),
            pl.BlockSpec((1, HID), lambda i: (0, 0)),
            pl.BlockSpec((1, HID), lambda i: (0, 0)),
        ],
        out_specs=[pl.BlockSpec((blk, HID), lambda i: (i, 0)),
                   pl.BlockSpec((blk, HID), lambda i: (i, 0))],
        out_shape=[jax.ShapeDtypeStruct((n, HID), jnp.float32),
                   jax.ShapeDtypeStruct((n, HID), jnp.bfloat16)],
    )(atom_fea, emb_w, emb_b.reshape(1, -1), g.reshape(1, -1), b.reshape(1, -1))


def _embed_edges(nbr_fea_p, edge_w, edge_b):
    e_pad, f = nbr_fea_p.shape
    blk = 4096
    grid = e_pad // blk

    def body(a_ref, w_ref, b_ref, o_ref):
        h = jnp.dot(a_ref[...], w_ref[...],
                    preferred_element_type=jnp.float32) + b_ref[...]
        o_ref[...] = _silu(h).astype(jnp.bfloat16)

    return pl.pallas_call(
        body,
        grid=(grid,),
        in_specs=[
            pl.BlockSpec((blk, f), lambda i: (i, 0)),
            pl.BlockSpec((f, HID), lambda i: (0, 0)),
            pl.BlockSpec((1, HID), lambda i: (0, 0)),
        ],
        out_specs=pl.BlockSpec((blk, HID), lambda i: (i, 0)),
        out_shape=jax.ShapeDtypeStruct((e_pad, HID), jnp.bfloat16),
    )(nbr_fea_p, edge_w, edge_b.reshape(1, -1))


def _conv_mlp(gx, ef, w1x, w1e, b1, w2, b2, hoff):
    e_half = gx.shape[0]
    blk = 4096
    grid = e_half // blk

    def body(gx_ref, ef_ref, w1x_ref, w1e_ref, b1_ref, w2_ref, b2_ref, o_ref):
        h = (jnp.dot(gx_ref[...], w1x_ref[...], preferred_element_type=jnp.float32)
             + jnp.dot(ef_ref[...], w1e_ref[...], preferred_element_type=jnp.float32)
             + b1_ref[...])
        h = _silu(h)
        m = jnp.dot(h, w2_ref[...], preferred_element_type=jnp.float32) + b2_ref[...]
        o_ref[...] = _silu(m)

    return pl.pallas_call(
        body,
        grid=(grid,),
        in_specs=[
            pl.BlockSpec((blk, HID), lambda i: (i, 0)),
            pl.BlockSpec((blk, HID), lambda i: (i + hoff, 0)),
            pl.BlockSpec((HID, 2 * HID), lambda i: (0, 0)),
            pl.BlockSpec((HID, 2 * HID), lambda i: (0, 0)),
            pl.BlockSpec((1, 2 * HID), lambda i: (0, 0)),
            pl.BlockSpec((2 * HID, HID), lambda i: (0, 0)),
            pl.BlockSpec((1, HID), lambda i: (0, 0)),
        ],
        out_specs=pl.BlockSpec((blk, HID), lambda i: (i, 0)),
        out_shape=jax.ShapeDtypeStruct((e_half, HID), jnp.float32),
    )(gx, ef, w1x, w1e, b1.reshape(1, -1), w2, b2.reshape(1, -1))


def _ln_residual(x, aggrs, cnt2, g, b):
    n = x.shape[0]
    blk = 2000
    grid = n // blk
    na = len(aggrs)

    def body(*refs):
        x_ref = refs[0]
        a_refs = refs[1:1 + na]
        ca_ref, cb_ref, g_ref, b_ref, o_ref, oh_ref = refs[1 + na:]
        cnt = ca_ref[0] + cb_ref[0]
        inv = 1.0 / jnp.maximum(cnt[:, 0:1], 1.0)
        agg = a_refs[0][...]
        for a in a_refs[1:]:
            agg = agg + a[...]
        y = x_ref[...] + agg * inv
        z = _ln(y, g_ref[...], b_ref[...])
        o_ref[...] = z
        oh_ref[...] = z.astype(jnp.bfloat16)

    return pl.pallas_call(
        body,
        grid=(grid,),
        in_specs=[
            pl.BlockSpec((blk, HID), lambda i: (i, 0)),
        ] + [
            pl.BlockSpec((blk, HID), lambda i: (i, 0)) for _ in range(na)
        ] + [
            pl.BlockSpec((1, blk, CW), lambda i: (0, i, 0)),
            pl.BlockSpec((1, blk, CW), lambda i: (1, i, 0)),
            pl.BlockSpec((1, HID), lambda i: (0, 0)),
            pl.BlockSpec((1, HID), lambda i: (0, 0)),
        ],
        out_specs=[pl.BlockSpec((blk, HID), lambda i: (i, 0)),
                   pl.BlockSpec((blk, HID), lambda i: (i, 0))],
        out_shape=[jax.ShapeDtypeStruct((n, HID), jnp.float32),
                   jax.ShapeDtypeStruct((n, HID), jnp.bfloat16)],
    )(x, *aggrs, cnt2, cnt2, g.reshape(1, -1), b.reshape(1, -1))


def _pool_head(x, bm3, w1, b1, w2, b2, w3p, b3p, n_graphs):
    n = x.shape[0]
    blk = 2000
    grid = n // blk

    def body(x_ref, bm_ref, w1_ref, b1_ref, w2_ref, b2_ref, w3_ref, b3_ref,
             o_ref, acc_ref):
        i = pl.program_id(0)

        @pl.when(i == 0)
        def _():
            acc_ref[...] = jnp.zeros_like(acc_ref)

        bm = bm_ref[...].reshape(1, blk)
        gid = lax.broadcasted_iota(jnp.int32, (n_graphs, blk), 0)
        oh = (gid == bm).astype(jnp.float32)
        xa = jnp.concatenate(
            [x_ref[...], jnp.ones((blk, HID), jnp.float32)], axis=1)
        acc_ref[...] += jnp.dot(oh, xa, preferred_element_type=jnp.float32)

        @pl.when(i == grid - 1)
        def _():
            acc = acc_ref[...]
            crystal = acc[:, :HID] / jnp.maximum(acc[:, HID:], 1.0)
            h = _silu(jnp.dot(crystal, w1_ref[...],
                              preferred_element_type=jnp.float32) + b1_ref[...])
            h = _silu(jnp.dot(h, w2_ref[...],
                              preferred_element_type=jnp.float32) + b2_ref[...])
            o_ref[...] = jnp.dot(h, w3_ref[...],
                                 preferred_element_type=jnp.float32) + b3_ref[...]

    return pl.pallas_call(
        body,
        grid=(grid,),
        in_specs=[
            pl.BlockSpec((blk, HID), lambda i: (i, 0)),
            pl.BlockSpec((1, 1, blk), lambda i: (i, 0, 0)),
            pl.BlockSpec((HID, HID), lambda i: (0, 0)),
            pl.BlockSpec((1, HID), lambda i: (0, 0)),
            pl.BlockSpec((HID, HID // 2), lambda i: (0, 0)),
            pl.BlockSpec((1, HID // 2), lambda i: (0, 0)),
            pl.BlockSpec((HID // 2, 128), lambda i: (0, 0)),
            pl.BlockSpec((1, 128), lambda i: (0, 0)),
        ],
        out_specs=pl.BlockSpec((n_graphs, 128), lambda i: (0, 0)),
        out_shape=jax.ShapeDtypeStruct((n_graphs, 128), jnp.float32),
        scratch_shapes=[pltpu.VMEM((n_graphs, 2 * HID), jnp.float32)],
    )(x, bm3, w1, b1.reshape(1, -1), w2, b2.reshape(1, -1), w3p, b3p)


def kernel(atom_fea, nbr_fea, nbr_idx, batch_mapping, params):
    n, _ = atom_fea.shape
    e = nbr_fea.shape[0]
    n_layers = 5
    n_graphs = 256

    nhalf = 1
    e_quant = nhalf * NW * CHUNK * KBUF
    e_pad = ((e + e_quant - 1) // e_quant) * e_quant
    e_half = e_pad // nhalf
    n_quant = NS * CHUNK
    n_pad = ((n + n_quant - 1) // n_quant) * n_quant

    pad = e_pad - e
    nbr_fea_p = jnp.pad(nbr_fea, ((0, pad), (0, 0)))
    idx32 = nbr_idx.astype(jnp.int32)
    src_p = jnp.pad(idx32[:, 0], (0, pad))
    dst_p = jnp.pad(idx32[:, 1], (0, pad), constant_values=n_pad)

    gathers = [_make_gather(n, e_half, HID, h * e_half) for h in range(nhalf)]
    scatters = [_make_scatter(e_half, n_pad, HID, h * e_half) for h in range(nhalf)]
    count_k = _make_count(e_pad, n_pad)
    mlp_hoff = e_half // 4096

    zeros_blk = jnp.zeros((ZROWS, CW), jnp.float32)
    ones_blk = jnp.ones((CHUNK, CW), jnp.float32)

    x, xh = _embed_nodes(atom_fea, params['emb_W'], params['emb_b'],
                         params['emb_ln_g'], params['emb_ln_b'])
    ef = _embed_edges(nbr_fea_p, params['edge_W'], params['edge_b'])
    cnt2 = count_k(dst_p, zeros_blk, ones_blk)

    for l in range(n_layers):
        w1 = params[f'conv{l}_W1']
        aggr = []
        for h in range(nhalf):
            gx = gathers[h](xh, src_p)
            msgs = _conv_mlp(gx, ef,
                             w1[:HID].astype(jnp.bfloat16),
                             w1[HID:].astype(jnp.bfloat16),
                             params[f'conv{l}_b1'],
                             params[f'conv{l}_W2'], params[f'conv{l}_b2'],
                             h * mlp_hoff)
            aggr.append(scatters[h](msgs, dst_p, zeros_blk))
        x, xh = _ln_residual(x, aggr, cnt2,
                             params[f'ln{l}_g'], params[f'ln{l}_b'])

    bm3 = batch_mapping.astype(jnp.int32).reshape(n // 2000, 1, 2000)
    w3p = jnp.pad(params['out_W3'], ((0, 0), (0, 128 - params['out_W3'].shape[1])))
    b3p = jnp.pad(params['out_b3'], (0, 128 - params['out_b3'].shape[0])).reshape(1, -1)
    out = _pool_head(x, bm3, params['out_W1'], params['out_b1'],
                     params['out_W2'], params['out_b2'], w3p, b3p, n_graphs)
    return out[:, :params['out_W3'].shape[1]]


# speedup vs baseline: 1.4787x; 1.0031x over previous
import functools

import jax
import jax.numpy as jnp
from jax import lax
from jax.experimental import pallas as pl
from jax.experimental.pallas import tpu as pltpu
from jax.experimental.pallas import tpu_sc as plsc

NC, NS = 2, 16
NW = NC * NS
CHUNK = 128
HID = 64
LANES = 16


def _silu(x):
    return x * jax.nn.sigmoid(x)


def _ln(h, g, b, eps=1e-5):
    mu = jnp.mean(h, axis=-1, keepdims=True)
    var = jnp.mean((h - mu) ** 2, axis=-1, keepdims=True)
    return (h - mu) / jnp.sqrt(var + eps) * g + b


KBUF = 12


def _make_gather(n_rows, e_half, d, ebase, kg=18):
    per_w = e_half // NW
    n_chunks = per_w // CHUNK
    n_groups = n_chunks // kg
    assert n_chunks % kg == 0
    mesh = plsc.VectorSubcoreMesh(core_axis_name="c", subcore_axis_name="s")

    @functools.partial(
        pl.kernel,
        out_type=jax.ShapeDtypeStruct((e_half, d), jnp.bfloat16),
        mesh=mesh,
        compiler_params=pltpu.CompilerParams(use_tc_tiling_on_sc=False),
        scratch_types=[
            pltpu.VMEM((kg, CHUNK), jnp.int32),
            pltpu.VMEM((kg, CHUNK, d), jnp.bfloat16),
            pltpu.SemaphoreType.DMA,
            pltpu.SemaphoreType.DMA,
            pltpu.SemaphoreType.DMA,
        ],
    )
    def gather_k(x_hbm, idx_hbm, out_hbm, idx_v, rows_v, sem_i, sem_g, sem_o):
        wid = lax.axis_index("s") * NC + lax.axis_index("c")
        base = wid * per_w

        def outer(g, carry):
            off0 = base + g * (kg * CHUNK)
            hi = [pltpu.async_copy(idx_hbm.at[pl.ds(ebase + off0 + b * CHUNK, CHUNK)],
                                   idx_v.at[b], sem_i) for b in range(kg)]
            hg = []
            for b in range(kg):
                hi[b].wait()
                hg.append(pltpu.async_copy(x_hbm.at[idx_v.at[b]],
                                           rows_v.at[b], sem_g))
            ho = []
            for b in range(kg):
                hg[b].wait()
                ho.append(pltpu.async_copy(
                    rows_v.at[b], out_hbm.at[pl.ds(off0 + b * CHUNK, CHUNK)],
                    sem_o))
            for h in ho:
                h.wait()
            return carry

        lax.fori_loop(0, n_groups, outer, 0)

    return gather_k


CW = 16
ZROWS = 640


def _make_scatter(e_half, n_pad, d, ebase):
    per_tile = e_half // NS
    n_chunks = per_tile // CHUNK
    n_groups = n_chunks // KBUF
    assert per_tile % CHUNK == 0 and n_chunks % KBUF == 0
    assert (n_pad // NS) % 8 == 0
    acc_rows = n_pad + NS * CHUNK
    zcopies = acc_rows // NS // ZROWS
    assert acc_rows % (NS * ZROWS) == 0
    nrb = KBUF
    mesh = plsc.VectorSubcoreMesh(core_axis_name="c", subcore_axis_name="s")

    scratch = [
        pltpu.VMEM((KBUF, CHUNK), jnp.int32),
        pltpu.VMEM((nrb, CHUNK, CW), jnp.float32),
        pltpu.VMEM_SHARED((acc_rows, CW), jnp.float32),
        pltpu.SemaphoreType.DMA,
        pltpu.SemaphoreType.DMA,
        pltpu.SemaphoreType.DMA,
    ]

    @functools.partial(
        pl.kernel,
        out_type=jax.ShapeDtypeStruct((n_pad, d), jnp.float32),
        mesh=mesh,
        compiler_params=pltpu.CompilerParams(use_tc_tiling_on_sc=False),
        scratch_types=scratch,
    )
    def scatter_k(msg_hbm, dst_hbm, zeros_hbm, out_hbm,
                  idx_v, rows_v, acc, sem_d, sem_m, sem_s):
        cid = lax.axis_index("c")
        sid = lax.axis_index("s")

        for p in range(2):
            col0 = cid * (2 * CW) + p * CW

            def zero_body(i, carry):
                pltpu.sync_copy(zeros_hbm,
                                acc.at[pl.ds((sid * zcopies + i) * ZROWS, ZROWS)])
                return carry

            lax.fori_loop(0, zcopies, zero_body, 0)
            plsc.subcore_barrier()

            def body(g, carry):
                off0 = (sid * n_chunks + g * KBUF) * CHUNK
                hd = [pltpu.async_copy(
                    dst_hbm.at[pl.ds(ebase + off0 + b * CHUNK, CHUNK)],
                    idx_v.at[b], sem_d) for b in range(KBUF)]
                hm = [pltpu.async_copy(
                    msg_hbm.at[pl.ds(off0 + b * CHUNK, CHUNK), pl.ds(col0, CW)],
                    rows_v.at[b], sem_m) for b in range(KBUF)]
                pending = []
                for b in range(KBUF):
                    hd[b].wait()
                    hm[b].wait()
                    pending.append(pltpu.async_copy(
                        rows_v.at[b], acc.at[idx_v.at[b]], sem_s, add=True))
                for h in pending:
                    h.wait()
                return carry

            lax.fori_loop(0, n_groups, body, 0)
            plsc.subcore_barrier()

            stripe = n_pad // NS
            pltpu.sync_copy(
                acc.at[pl.ds(sid * stripe, stripe)],
                out_hbm.at[pl.ds(sid * stripe, stripe), pl.ds(col0, CW)])
            plsc.subcore_barrier()

    return scatter_k


def _make_count(e_pad, n_pad):
    per_tile = e_pad // NW
    n_chunks = per_tile // CHUNK
    n_groups = n_chunks // KBUF
    acc_rows = n_pad + NS * CHUNK
    zcopies = acc_rows // NS // ZROWS
    mesh = plsc.VectorSubcoreMesh(core_axis_name="c", subcore_axis_name="s")

    scratch = [
        pltpu.VMEM((KBUF, CHUNK), jnp.int32),
        pltpu.VMEM((CHUNK, CW), jnp.float32),
        pltpu.VMEM_SHARED((acc_rows, CW), jnp.float32),
        pltpu.SemaphoreType.DMA,
        pltpu.SemaphoreType.DMA,
    ]

    @functools.partial(
        pl.kernel,
        out_type=jax.ShapeDtypeStruct((2, n_pad, CW), jnp.float32),
        mesh=mesh,
        compiler_params=pltpu.CompilerParams(use_tc_tiling_on_sc=False),
        scratch_types=scratch,
    )
    def count_k(dst_hbm, zeros_hbm, ones_hbm, out_hbm,
                idx_v, ones_v, acc, sem_d, sem_s):
        cid = lax.axis_index("c")
        sid = lax.axis_index("s")
        pltpu.sync_copy(ones_hbm, ones_v)

        def zero_body(i, carry):
            pltpu.sync_copy(zeros_hbm,
                            acc.at[pl.ds((sid * zcopies + i) * ZROWS, ZROWS)])
            return carry

        lax.fori_loop(0, zcopies, zero_body, 0)
        plsc.subcore_barrier()

        def body(g, carry):
            off0 = ((cid * NS + sid) * n_chunks + g * KBUF) * CHUNK
            hd = [pltpu.async_copy(
                dst_hbm.at[pl.ds(off0 + b * CHUNK, CHUNK)],
                idx_v.at[b], sem_d) for b in range(KBUF)]
            pending = []
            for b in range(KBUF):
                hd[b].wait()
                pending.append(pltpu.async_copy(
                    ones_v, acc.at[idx_v.at[b]], sem_s, add=True))
            for h in pending:
                h.wait()
            return carry

        lax.fori_loop(0, n_groups, body, 0)
        plsc.subcore_barrier()

        stripe = n_pad // NS
        pltpu.sync_copy(acc.at[pl.ds(sid * stripe, stripe)],
                        out_hbm.at[cid, pl.ds(sid * stripe, stripe)])
        plsc.subcore_barrier()

    return count_k


def _embed_nodes(atom_fea, emb_w, emb_b, g, b):
    n, f = atom_fea.shape
    blk = 2000
    grid = n // blk

    def body(a_ref, w_ref, b_ref, g_ref, bb_ref, o_ref, oh_ref):
        h = jnp.dot(a_ref[...], w_ref[...],
                    preferred_element_type=jnp.float32) + b_ref[...]
        y = _silu(_ln(h, g_ref[...], bb_ref[...]))
        o_ref[...] = y
        oh_ref[...] = y.astype(jnp.bfloat16)

    return pl.pallas_call(
        body,
        grid=(grid,),
        in_specs=[
            pl.BlockSpec((blk, f), lambda i: (i, 0)),
            pl.BlockSpec((f, HID), lambda i: (0, 0)),
            pl.BlockSpec((1, HID), lambda i: (0, 0)),
            pl.BlockSpec((1, HID), lambda i: (0, 0)),
            pl.BlockSpec((1, HID), lambda i: (0, 0)),
        ],
        out_specs=[pl.BlockSpec((blk, HID), lambda i: (i, 0)),
                   pl.BlockSpec((blk, HID), lambda i: (i, 0))],
        out_shape=[jax.ShapeDtypeStruct((n, HID), jnp.float32),
                   jax.ShapeDtypeStruct((n, HID), jnp.bfloat16)],
    )(atom_fea, emb_w, emb_b.reshape(1, -1), g.reshape(1, -1), b.reshape(1, -1))


def _embed_edges(nbr_fea_p, edge_w, edge_b):
    e_pad, f = nbr_fea_p.shape
    blk = 4096
    grid = e_pad // blk

    def body(a_ref, w_ref, b_ref, o_ref):
        h = jnp.dot(a_ref[...], w_ref[...],
                    preferred_element_type=jnp.float32) + b_ref[...]
        o_ref[...] = _silu(h).astype(jnp.bfloat16)

    return pl.pallas_call(
        body,
        grid=(grid,),
        in_specs=[
            pl.BlockSpec((blk, f), lambda i: (i, 0)),
            pl.BlockSpec((f, HID), lambda i: (0, 0)),
            pl.BlockSpec((1, HID), lambda i: (0, 0)),
        ],
        out_specs=pl.BlockSpec((blk, HID), lambda i: (i, 0)),
        out_shape=jax.ShapeDtypeStruct((e_pad, HID), jnp.bfloat16),
    )(nbr_fea_p, edge_w, edge_b.reshape(1, -1))


def _conv_mlp(gx, ef, w1x, w1e, b1, w2, b2, hoff):
    e_half = gx.shape[0]
    blk = 4096
    grid = e_half // blk

    def body(gx_ref, ef_ref, w1x_ref, w1e_ref, b1_ref, w2_ref, b2_ref, o_ref):
        h = (jnp.dot(gx_ref[...], w1x_ref[...], preferred_element_type=jnp.float32)
             + jnp.dot(ef_ref[...], w1e_ref[...], preferred_element_type=jnp.float32)
             + b1_ref[...])
        h = _silu(h)
        m = jnp.dot(h, w2_ref[...], preferred_element_type=jnp.float32) + b2_ref[...]
        o_ref[...] = _silu(m)

    return pl.pallas_call(
        body,
        grid=(grid,),
        in_specs=[
            pl.BlockSpec((blk, HID), lambda i: (i, 0)),
            pl.BlockSpec((blk, HID), lambda i: (i + hoff, 0)),
            pl.BlockSpec((HID, 2 * HID), lambda i: (0, 0)),
            pl.BlockSpec((HID, 2 * HID), lambda i: (0, 0)),
            pl.BlockSpec((1, 2 * HID), lambda i: (0, 0)),
            pl.BlockSpec((2 * HID, HID), lambda i: (0, 0)),
            pl.BlockSpec((1, HID), lambda i: (0, 0)),
        ],
        out_specs=pl.BlockSpec((blk, HID), lambda i: (i, 0)),
        out_shape=jax.ShapeDtypeStruct((e_half, HID), jnp.float32),
    )(gx, ef, w1x, w1e, b1.reshape(1, -1), w2, b2.reshape(1, -1))


def _ln_residual(x, aggrs, cnt2, g, b):
    n = x.shape[0]
    blk = 2000
    grid = n // blk
    na = len(aggrs)

    def body(*refs):
        x_ref = refs[0]
        a_refs = refs[1:1 + na]
        ca_ref, cb_ref, g_ref, b_ref, o_ref, oh_ref = refs[1 + na:]
        cnt = ca_ref[0] + cb_ref[0]
        inv = 1.0 / jnp.maximum(cnt[:, 0:1], 1.0)
        agg = a_refs[0][...]
        for a in a_refs[1:]:
            agg = agg + a[...]
        y = x_ref[...] + agg * inv
        z = _ln(y, g_ref[...], b_ref[...])
        o_ref[...] = z
        oh_ref[...] = z.astype(jnp.bfloat16)

    return pl.pallas_call(
        body,
        grid=(grid,),
        in_specs=[
            pl.BlockSpec((blk, HID), lambda i: (i, 0)),
        ] + [
            pl.BlockSpec((blk, HID), lambda i: (i, 0)) for _ in range(na)
        ] + [
            pl.BlockSpec((1, blk, CW), lambda i: (0, i, 0)),
            pl.BlockSpec((1, blk, CW), lambda i: (1, i, 0)),
            pl.BlockSpec((1, HID), lambda i: (0, 0)),
            pl.BlockSpec((1, HID), lambda i: (0, 0)),
        ],
        out_specs=[pl.BlockSpec((blk, HID), lambda i: (i, 0)),
                   pl.BlockSpec((blk, HID), lambda i: (i, 0))],
        out_shape=[jax.ShapeDtypeStruct((n, HID), jnp.float32),
                   jax.ShapeDtypeStruct((n, HID), jnp.bfloat16)],
    )(x, *aggrs, cnt2, cnt2, g.reshape(1, -1), b.reshape(1, -1))


def _pool_head(x, bm3, w1, b1, w2, b2, w3p, b3p, n_graphs):
    n = x.shape[0]
    blk = 2000
    grid = n // blk

    def body(x_ref, bm_ref, w1_ref, b1_ref, w2_ref, b2_ref, w3_ref, b3_ref,
             o_ref, acc_ref):
        i = pl.program_id(0)

        @pl.when(i == 0)
        def _():
            acc_ref[...] = jnp.zeros_like(acc_ref)

        bm = bm_ref[...].reshape(1, blk)
        gid = lax.broadcasted_iota(jnp.int32, (n_graphs, blk), 0)
        oh = (gid == bm).astype(jnp.float32)
        xa = jnp.concatenate(
            [x_ref[...], jnp.ones((blk, HID), jnp.float32)], axis=1)
        acc_ref[...] += jnp.dot(oh, xa, preferred_element_type=jnp.float32)

        @pl.when(i == grid - 1)
        def _():
            acc = acc_ref[...]
            crystal = acc[:, :HID] / jnp.maximum(acc[:, HID:], 1.0)
            h = _silu(jnp.dot(crystal, w1_ref[...],
                              preferred_element_type=jnp.float32) + b1_ref[...])
            h = _silu(jnp.dot(h, w2_ref[...],
                              preferred_element_type=jnp.float32) + b2_ref[...])
            o_ref[...] = jnp.dot(h, w3_ref[...],
                                 preferred_element_type=jnp.float32) + b3_ref[...]

    return pl.pallas_call(
        body,
        grid=(grid,),
        in_specs=[
            pl.BlockSpec((blk, HID), lambda i: (i, 0)),
            pl.BlockSpec((1, 1, blk), lambda i: (i, 0, 0)),
            pl.BlockSpec((HID, HID), lambda i: (0, 0)),
            pl.BlockSpec((1, HID), lambda i: (0, 0)),
            pl.BlockSpec((HID, HID // 2), lambda i: (0, 0)),
            pl.BlockSpec((1, HID // 2), lambda i: (0, 0)),
            pl.BlockSpec((HID // 2, 128), lambda i: (0, 0)),
            pl.BlockSpec((1, 128), lambda i: (0, 0)),
        ],
        out_specs=pl.BlockSpec((n_graphs, 128), lambda i: (0, 0)),
        out_shape=jax.ShapeDtypeStruct((n_graphs, 128), jnp.float32),
        scratch_shapes=[pltpu.VMEM((n_graphs, 2 * HID), jnp.float32)],
    )(x, bm3, w1, b1.reshape(1, -1), w2, b2.reshape(1, -1), w3p, b3p)


def kernel(atom_fea, nbr_fea, nbr_idx, batch_mapping, params):
    n, _ = atom_fea.shape
    e = nbr_fea.shape[0]
    n_layers = 5
    n_graphs = 256

    nhalf = 1
    e_quant = nhalf * NW * CHUNK * KBUF
    e_pad = ((e + e_quant - 1) // e_quant) * e_quant
    e_half = e_pad // nhalf
    n_quant = NS * CHUNK
    n_pad = ((n + n_quant - 1) // n_quant) * n_quant

    pad = e_pad - e
    nbr_fea_p = jnp.pad(nbr_fea, ((0, pad), (0, 0)))
    idx32 = nbr_idx.astype(jnp.int32)
    src_p = jnp.pad(idx32[:, 0], (0, pad))
    dst_p = jnp.pad(idx32[:, 1], (0, pad), constant_values=n_pad)

    gathers = [_make_gather(n, e_half, HID, h * e_half) for h in range(nhalf)]
    scatters = [_make_scatter(e_half, n_pad, HID, h * e_half) for h in range(nhalf)]
    count_k = _make_count(e_pad, n_pad)
    mlp_hoff = e_half // 4096

    zeros_blk = jnp.zeros((ZROWS, CW), jnp.float32)
    ones_blk = jnp.ones((CHUNK, CW), jnp.float32)

    x, xh = _embed_nodes(atom_fea, params['emb_W'], params['emb_b'],
                         params['emb_ln_g'], params['emb_ln_b'])
    ef = _embed_edges(nbr_fea_p, params['edge_W'], params['edge_b'])
    cnt2 = count_k(dst_p, zeros_blk, ones_blk)

    for l in range(n_layers):
        w1 = params[f'conv{l}_W1']
        aggr = []
        for h in range(nhalf):
            gx = gathers[h](xh, src_p)
            msgs = _conv_mlp(gx, ef,
                             w1[:HID].astype(jnp.bfloat16),
                             w1[HID:].astype(jnp.bfloat16),
                             params[f'conv{l}_b1'],
                             params[f'conv{l}_W2'], params[f'conv{l}_b2'],
                             h * mlp_hoff)
            aggr.append(scatters[h](msgs, dst_p, zeros_blk))
        x, xh = _ln_residual(x, aggr, cnt2,
                             params[f'ln{l}_g'], params[f'ln{l}_b'])

    bm3 = batch_mapping.astype(jnp.int32).reshape(n // 2000, 1, 2000)
    w3p = jnp.pad(params['out_W3'], ((0, 0), (0, 128 - params['out_W3'].shape[1])))
    b3p = jnp.pad(params['out_b3'], (0, 128 - params['out_b3'].shape[0])).reshape(1, -1)
    out = _pool_head(x, bm3, params['out_W1'], params['out_b1'],
                     params['out_W2'], params['out_b2'], w3p, b3p, n_graphs)
    return out[:, :params['out_W3'].shape[1]]
